# baseline (speedup 1.0000x reference)
import jax
import jax.numpy as jnp
from jax import lax
from jax.experimental import pallas as pl
from jax.experimental.pallas import tpu as pltpu
from jax.experimental.pallas import tpu_sc as plsc

B, S, D = 4, 4096, 1024
BSC = 2
BTC = B - BSC
NC, NS = 2, 16
NW = NC * NS
SPW = S // NW
C = 8
G = SPW // C
NB = 6
NP = 3
LOOK = 2


def _sc_body(emb_hbm, pos_hbm, out_hbm, *refs):
    ebufs = [[refs[k * BSC + b] for b in range(BSC)] for k in range(NB)]
    pbufs = list(refs[NB * BSC:NB * BSC + NP])
    sems = refs[NB * BSC + NP:]
    se = sems[:NB]
    so = sems[NB:2 * NB]
    sp = sems[2 * NB:2 * NB + NP]

    wid = lax.axis_index("s") * NC + lax.axis_index("c")
    seq_base = wid * SPW

    de, dp, do = {}, {}, {}

    def issue_in(i):
        k = i % NB
        row = seq_base + i * C
        de[i] = [
            pltpu.async_copy(
                emb_hbm.at[b, pl.ds(row, C), :], ebufs[k][b], se[k])
            for b in range(BSC)
        ]
        dp[i] = pltpu.async_copy(
            pos_hbm.at[pl.ds(row, C), :], pbufs[i % NP], sp[i % NP])

    for i in range(LOOK):
        issue_in(i)

    for i in range(G):
        k = i % NB
        if i >= LOOK:
            for d in do[i - LOOK]:
                d.wait()
        if i + LOOK < G:
            issue_in(i + LOOK)
        for d in de[i]:
            d.wait()
        dp[i].wait()
        eb, pb = ebufs[k], pbufs[i % NP]

        def row_add(r, carry):
            @plsc.parallel_loop(0, D, 16, unroll=4)
            def _(c):
                sl = pl.ds(c, 16)
                pv = pb[r, sl]
                for b in range(BSC):
                    eb[b][r, sl] = eb[b][r, sl] + pv
            return carry

        lax.fori_loop(0, C, row_add, 0)

        row = seq_base + i * C
        do[i] = [
            pltpu.async_copy(
                ebufs[k][b], out_hbm.at[b, pl.ds(row, C), :], so[k])
            for b in range(BSC)
        ]

    for i in range(max(0, G - LOOK), G):
        for d in do[i]:
            d.wait()


def _tc_body(emb_ref, pos_ref, out_ref):
    out_ref[...] = emb_ref[...] + pos_ref[...][None, :, :]


TC_BLK = 512


@jax.jit
def _run(embeddings, pos_table):
    sc = pl.kernel(
        _sc_body,
        out_type=jax.ShapeDtypeStruct((BSC, S, D), jnp.float32),
        mesh=plsc.VectorSubcoreMesh(
            core_axis_name="c", subcore_axis_name="s",
            num_cores=NC, num_subcores=NS,
        ),
        scratch_types=(
            [pltpu.VMEM((C, D), jnp.float32)] * (NB * BSC + NP)
            + [pltpu.SemaphoreType.DMA] * (2 * NB + NP)
        ),
        compiler_params=pltpu.CompilerParams(use_tc_tiling_on_sc=True),
    )
    sc_part = sc(embeddings[:BSC], pos_table)

    tc_part = pl.pallas_call(
        _tc_body,
        out_shape=jax.ShapeDtypeStruct((BTC, S, D), jnp.float32),
        grid=(BTC, S // TC_BLK),
        in_specs=[
            pl.BlockSpec((1, TC_BLK, D), lambda b, i: (b, i, 0)),
            pl.BlockSpec((TC_BLK, D), lambda b, i: (i, 0)),
        ],
        out_specs=pl.BlockSpec((1, TC_BLK, D), lambda b, i: (b, i, 0)),
    )(embeddings[BSC:], pos_table[:S])

    return jnp.concatenate([sc_part, tc_part], axis=0)


def kernel(embeddings, pos_table):
    return _run(embeddings, pos_table)


# speedup vs baseline: 2.2542x; 2.2542x over previous
import jax
import jax.numpy as jnp
from jax import lax
from jax.experimental import pallas as pl
from jax.experimental.pallas import tpu as pltpu
from jax.experimental.pallas import tpu_sc as plsc

B, S, D = 4, 4096, 1024
NC, NS = 2, 16
NW = NC * NS
SPW = S // NW
C = 8
G = SPW // C
NPAIR = 2
NPG = B // NPAIR
NI = G * NPG
NB = 6
NP = 3
LOOK = 2


def _pos_add_body(emb_hbm, pos_hbm, out_hbm, *refs):
    ebufs = [[refs[k * NPAIR + b] for b in range(NPAIR)] for k in range(NB)]
    pbufs = list(refs[NB * NPAIR:NB * NPAIR + NP])
    sems = refs[NB * NPAIR + NP:]
    se = sems[:NB]
    so = sems[NB:2 * NB]
    sp = sems[2 * NB:2 * NB + NP]

    wid = lax.axis_index("s") * NC + lax.axis_index("c")
    seq_base = wid * SPW

    de, dp, do = {}, {}, {}

    def issue_in(i):
        g, p = divmod(i, NPG)
        k = i % NB
        row = seq_base + g * C
        de[i] = [
            pltpu.async_copy(
                emb_hbm.at[p * NPAIR + b, pl.ds(row, C), :],
                ebufs[k][b], se[k])
            for b in range(NPAIR)
        ]
        if p == 0:
            dp[g] = pltpu.async_copy(
                pos_hbm.at[pl.ds(row, C), :], pbufs[g % NP], sp[g % NP])

    for i in range(LOOK):
        issue_in(i)

    for i in range(NI):
        g, p = divmod(i, NPG)
        k = i % NB
        if i >= LOOK:
            for d in do[i - LOOK]:
                d.wait()
        if i + LOOK < NI:
            issue_in(i + LOOK)
        for d in de[i]:
            d.wait()
        if p == 0:
            dp[g].wait()
        eb, pb = ebufs[k], pbufs[g % NP]

        def row_add(r, carry):
            @plsc.parallel_loop(0, D, 16, unroll=4)
            def _(c):
                sl = pl.ds(c, 16)
                pv = pb[r, sl]
                for b in range(NPAIR):
                    eb[b][r, sl] = eb[b][r, sl] + pv
            return carry

        lax.fori_loop(0, C, row_add, 0)

        row = seq_base + g * C
        do[i] = [
            pltpu.async_copy(
                ebufs[k][b],
                out_hbm.at[p * NPAIR + b, pl.ds(row, C), :], so[k])
            for b in range(NPAIR)
        ]

    for i in range(max(0, NI - LOOK), NI):
        for d in do[i]:
            d.wait()


@jax.jit
def _run(embeddings, pos_table):
    f = pl.kernel(
        _pos_add_body,
        out_type=jax.ShapeDtypeStruct((B, S, D), jnp.float32),
        mesh=plsc.VectorSubcoreMesh(
            core_axis_name="c", subcore_axis_name="s",
            num_cores=NC, num_subcores=NS,
        ),
        scratch_types=(
            [pltpu.VMEM((C, D), jnp.float32)] * (NB * NPAIR + NP)
            + [pltpu.SemaphoreType.DMA] * (2 * NB + NP)
        ),
        compiler_params=pltpu.CompilerParams(use_tc_tiling_on_sc=True),
    )
    return f(embeddings, pos_table)


def kernel(embeddings, pos_table):
    return _run(embeddings, pos_table)


# speedup vs baseline: 2.2577x; 1.0015x over previous
import jax
import jax.numpy as jnp
from jax import lax
from jax.experimental import pallas as pl
from jax.experimental.pallas import tpu as pltpu
from jax.experimental.pallas import tpu_sc as plsc

B, S, D = 4, 4096, 1024
NC, NS = 2, 16
NW = NC * NS
SPW = S // NW
C = 8
G = SPW // C
NPAIR = 2
NPG = B // NPAIR
NI = G * NPG
NB = 6
NP = 3
LOOK = 3


def _pos_add_body(emb_hbm, pos_hbm, out_hbm, *refs):
    ebufs = [[refs[k * NPAIR + b] for b in range(NPAIR)] for k in range(NB)]
    pbufs = list(refs[NB * NPAIR:NB * NPAIR + NP])
    sems = refs[NB * NPAIR + NP:]
    se = sems[:NB]
    so = sems[NB:2 * NB]
    sp = sems[2 * NB:2 * NB + NP]

    wid = lax.axis_index("s") * NC + lax.axis_index("c")
    seq_base = wid * SPW

    de, dp, do = {}, {}, {}

    def issue_in(i):
        g, p = divmod(i, NPG)
        k = i % NB
        row = seq_base + g * C
        de[i] = [
            pltpu.async_copy(
                emb_hbm.at[p * NPAIR + b, pl.ds(row, C), :],
                ebufs[k][b], se[k])
            for b in range(NPAIR)
        ]
        if p == 0:
            dp[g] = pltpu.async_copy(
                pos_hbm.at[pl.ds(row, C), :], pbufs[g % NP], sp[g % NP])

    for i in range(LOOK):
        issue_in(i)

    for i in range(NI):
        g, p = divmod(i, NPG)
        k = i % NB
        if i >= LOOK:
            for d in do[i - LOOK]:
                d.wait()
        if i + LOOK < NI:
            issue_in(i + LOOK)
        for d in de[i]:
            d.wait()
        if p == 0:
            dp[g].wait()
        eb, pb = ebufs[k], pbufs[g % NP]

        def row_add(r, carry):
            @plsc.parallel_loop(0, D, 16, unroll=4)
            def _(c):
                sl = pl.ds(c, 16)
                pv = pb[r, sl]
                for b in range(NPAIR):
                    eb[b][r, sl] = eb[b][r, sl] + pv
            return carry

        lax.fori_loop(0, C, row_add, 0)

        row = seq_base + g * C
        do[i] = [
            pltpu.async_copy(
                ebufs[k][b],
                out_hbm.at[p * NPAIR + b, pl.ds(row, C), :], so[k])
            for b in range(NPAIR)
        ]

    for i in range(max(0, NI - LOOK), NI):
        for d in do[i]:
            d.wait()


@jax.jit
def _run(embeddings, pos_table):
    f = pl.kernel(
        _pos_add_body,
        out_type=jax.ShapeDtypeStruct((B, S, D), jnp.float32),
        mesh=plsc.VectorSubcoreMesh(
            core_axis_name="c", subcore_axis_name="s",
            num_cores=NC, num_subcores=NS,
        ),
        scratch_types=(
            [pltpu.VMEM((C, D), jnp.float32)] * (NB * NPAIR + NP)
            + [pltpu.SemaphoreType.DMA] * (2 * NB + NP)
        ),
        compiler_params=pltpu.CompilerParams(use_tc_tiling_on_sc=True),
    )
    return f(embeddings, pos_table)


def kernel(embeddings, pos_table):
    return _run(embeddings, pos_table)
